# baseline (speedup 1.0000x reference)
import functools

import jax
import jax.numpy as jnp
from jax import lax
from jax.experimental import pallas as pl
from jax.experimental.pallas import tpu as pltpu
from jax.experimental.pallas import tpu_sc as plsc

N = 10000
E = 320000
IN_DIM = 128
HID = 64
OUT = 64
EDGE_DIM = 16
XP = 16
TW = HID + XP
EPS = 1e-8

NC = 2
NS = 16
NW = NC * NS
EW = E // NW
CHUNK = 100
NCHUNK = EW // CHUNK
KB = 5
NBURST = NCHUNK // KB

ROWS_PT = N // NS


def _worker_id():
    return lax.axis_index("s") * NC + lax.axis_index("c")


@functools.cache
def _sc_kernels():
    mesh = plsc.VectorSubcoreMesh(
        core_axis_name="c", subcore_axis_name="s",
        num_cores=NC, num_subcores=NS)
    sc_params = pltpu.CompilerParams(use_tc_tiling_on_sc=False)

    @functools.partial(
        pl.kernel,
        out_type=(
            jax.ShapeDtypeStruct((E, TW), jnp.float32),
            jax.ShapeDtypeStruct((E, TW), jnp.float32),
        ),
        mesh=mesh,
        scratch_types=(
            pltpu.VMEM((NCHUNK, CHUNK), jnp.int32),
            pltpu.VMEM((NCHUNK, CHUNK), jnp.int32),
            pltpu.VMEM((KB, CHUNK, TW), jnp.float32),
            pltpu.VMEM((KB, CHUNK, TW), jnp.float32),
            pltpu.SemaphoreType.DMA,
            pltpu.SemaphoreType.DMA,
        ),
        compiler_params=sc_params,
    )
    def sc_gather(thx_hbm, row_hbm, col_hbm, a_hbm, b_hbm,
                  row_v, col_v, ab, bb, gsem, wsem):
        wid = _worker_id()
        base = wid * EW
        pltpu.sync_copy(row_hbm.at[wid], row_v)
        pltpu.sync_copy(col_hbm.at[wid], col_v)

        def body(g, carry):
            j0 = g * KB
            gcp = []
            for b in range(KB):
                gcp.append(pltpu.async_copy(
                    thx_hbm.at[row_v.at[j0 + b]], ab.at[b], gsem))
                gcp.append(pltpu.async_copy(
                    thx_hbm.at[col_v.at[j0 + b]], bb.at[b], gsem))
            wcp = []
            for b in range(KB):
                gcp[2 * b].wait()
                gcp[2 * b + 1].wait()
                sl = pl.ds(base + (j0 + b) * CHUNK, CHUNK)
                wcp.append(pltpu.async_copy(ab.at[b], a_hbm.at[sl], wsem))
                wcp.append(pltpu.async_copy(bb.at[b], b_hbm.at[sl], wsem))
            for c in wcp:
                c.wait()
            return carry

        lax.fori_loop(0, NBURST, body, None)

    @functools.partial(
        pl.kernel,
        out_type=jax.ShapeDtypeStruct((NC, N, TW), jnp.float32),
        mesh=mesh,
        scratch_types=(
            pltpu.VMEM((NCHUNK, CHUNK), jnp.int32),
            pltpu.VMEM((KB, CHUNK, TW), jnp.float32),
            pltpu.VMEM_SHARED((N, TW), jnp.float32),
            pltpu.SemaphoreType.DMA,
            pltpu.SemaphoreType.DMA,
        ),
        compiler_params=sc_params,
    )
    def sc_scatter(mt_hbm, row_hbm, z80_hbm, acc_hbm,
                   row_v, mb, acc_sh, lsem, ssem):
        cid = lax.axis_index("c")
        sid = lax.axis_index("s")
        wid = _worker_id()
        base = wid * EW
        rsl = pl.ds(sid * ROWS_PT, ROWS_PT)
        pltpu.sync_copy(z80_hbm.at[rsl], acc_sh.at[rsl])
        pltpu.sync_copy(row_hbm.at[wid], row_v)
        plsc.subcore_barrier()

        def body(g, carry):
            j0 = g * KB
            lcp = []
            for b in range(KB):
                sl = pl.ds(base + (j0 + b) * CHUNK, CHUNK)
                lcp.append(pltpu.async_copy(mt_hbm.at[sl], mb.at[b], lsem))
            scp = []
            for b in range(KB):
                lcp[b].wait()
                scp.append(pltpu.async_copy(
                    mb.at[b], acc_sh.at[row_v.at[j0 + b]], ssem, add=True))
            for c in scp:
                c.wait()
            return carry

        lax.fori_loop(0, NBURST, body, None)
        plsc.subcore_barrier()
        pltpu.sync_copy(acc_sh.at[rsl], acc_hbm.at[cid].at[rsl])

    return sc_gather, sc_scatter


def _sc_gather(thx, row3, col3):
    return _sc_kernels()[0](thx, row3, col3)


def _sc_scatter(mt, row3, z80):
    return _sc_kernels()[1](mt, row3, z80)


BE = 2000
BN = 1000


def _silu(v):
    return v * jax.nn.sigmoid(v)


def _edge_body(a, b, ea,
               w1a, w1b, w1c, w1d, be1, we2, be2, watt, batt, wc1, bc1, wc2,
               mt_out):
    hi = a[:, :HID]
    hj = b[:, :HID]
    d = a[:, HID:] - b[:, HID:]
    radial = jnp.sum(d * d, axis=-1, keepdims=True)
    dn = d / (jnp.sqrt(radial) + EPS)
    t = jnp.dot(hi, w1a[...], preferred_element_type=jnp.float32)
    t += jnp.dot(hj, w1b[...], preferred_element_type=jnp.float32)
    t += jnp.dot(ea[...], w1d[...], preferred_element_type=jnp.float32)
    t += radial * w1c[...] + be1[...]
    m1 = _silu(t)
    m2 = _silu(jnp.dot(m1, we2[...], preferred_element_type=jnp.float32) + be2[...])
    att = jax.nn.sigmoid(
        jnp.dot(m2, watt[...], preferred_element_type=jnp.float32) + batt[...])
    m = m2 * att
    s = jnp.dot(_silu(jnp.dot(m, wc1[...], preferred_element_type=jnp.float32)
                      + bc1[...]),
                wc2[...], preferred_element_type=jnp.float32)
    tp = dn * s
    colpos = lax.broadcasted_iota(jnp.int32, tp.shape, 1)
    tp = jnp.where(colpos == 3, 1.0, tp)
    mt_out[...] = jnp.concatenate([m, tp], axis=1)


def _edge_tc(a, b, ea, w1a, w1b, w1c, w1d, be1, we2, be2,
             watt, batt, wc1, bc1, wc2):
    grid = (E // BE,)

    def eb(i):
        return (i, 0)

    def zb(i):
        return (0, 0)

    full = lambda arr: pl.BlockSpec(arr.shape, zb)
    return pl.pallas_call(
        _edge_body,
        grid=grid,
        in_specs=[
            pl.BlockSpec((BE, TW), eb),
            pl.BlockSpec((BE, TW), eb),
            pl.BlockSpec((BE, EDGE_DIM), eb),
            full(w1a), full(w1b), full(w1c), full(w1d), full(be1),
            full(we2), full(be2), full(watt), full(batt),
            full(wc1), full(bc1), full(wc2),
        ],
        out_specs=pl.BlockSpec((BE, TW), eb),
        out_shape=jax.ShapeDtypeStruct((E, TW), jnp.float32),
        compiler_params=pltpu.CompilerParams(
            dimension_semantics=("arbitrary",)),
    )(a, b, ea, w1a, w1b, w1c, w1d, be1, we2, be2, watt, batt, wc1, bc1, wc2)


def _node_body(thx, accp, wn1a, wn1b, bn1, wn2, bn2, g, b, thx_out):
    h = thx[:, :HID]
    xp = thx[:, HID:]
    acc = accp[0] + accp[1]
    agg = acc[:, :HID]
    xs = acc[:, HID:]
    cnt = jnp.maximum(xs[:, 3:4], 1.0)
    xnew = xp + xs / cnt
    colpos = lax.broadcasted_iota(jnp.int32, xnew.shape, 1)
    xnew = jnp.where(colpos < 3, xnew, 0.0)
    u = _silu(jnp.dot(h, wn1a[...], preferred_element_type=jnp.float32)
              + jnp.dot(agg, wn1b[...], preferred_element_type=jnp.float32)
              + bn1[...])
    hu = jnp.dot(u, wn2[...], preferred_element_type=jnp.float32) + bn2[...]
    hn = h + hu
    mu = jnp.mean(hn, axis=-1, keepdims=True)
    var = jnp.mean((hn - mu) ** 2, axis=-1, keepdims=True)
    ho = (hn - mu) / jnp.sqrt(var + 1e-5) * g[...] + b[...]
    thx_out[...] = jnp.concatenate([ho, xnew], axis=1)


def _node_tc(thx, accp, wn1a, wn1b, bn1, wn2, bn2, g, b):
    grid = (N // BN,)

    def nb(i):
        return (i, 0)

    def pb(i):
        return (0, i, 0)

    def zb(i):
        return (0, 0)

    full = lambda arr: pl.BlockSpec(arr.shape, zb)
    return pl.pallas_call(
        _node_body,
        grid=grid,
        in_specs=[
            pl.BlockSpec((BN, TW), nb),
            pl.BlockSpec((NC, BN, TW), pb),
            full(wn1a), full(wn1b), full(bn1), full(wn2), full(bn2),
            full(g), full(b),
        ],
        out_specs=pl.BlockSpec((BN, TW), nb),
        out_shape=jax.ShapeDtypeStruct((N, TW), jnp.float32),
        compiler_params=pltpu.CompilerParams(
            dimension_semantics=("arbitrary",)),
    )(thx, accp, wn1a, wn1b, bn1, wn2, bn2, g, b)


def _embed_body(h, xp, w, bias, o):
    hh = jnp.dot(h[...], w[...], preferred_element_type=jnp.float32) + bias[...]
    o[...] = jnp.concatenate([hh, xp[...]], axis=1)


def _embed_tc(h, xpad, w, bias):
    grid = (N // BN,)

    def nb(i):
        return (i, 0)

    def zb(i):
        return (0, 0)

    return pl.pallas_call(
        _embed_body,
        grid=grid,
        in_specs=[
            pl.BlockSpec((BN, IN_DIM), nb),
            pl.BlockSpec((BN, XP), nb),
            pl.BlockSpec(w.shape, zb),
            pl.BlockSpec(bias.shape, zb),
        ],
        out_specs=pl.BlockSpec((BN, TW), nb),
        out_shape=jax.ShapeDtypeStruct((N, TW), jnp.float32),
        compiler_params=pltpu.CompilerParams(
            dimension_semantics=("arbitrary",)),
    )(h, xpad, w, bias)


def _out_body(thx, w, bias, o):
    o[...] = jnp.dot(thx[:, :HID], w[...],
                     preferred_element_type=jnp.float32) + bias[...]


def _out_tc(thx, w, bias):
    grid = (N // BN,)

    def nb(i):
        return (i, 0)

    def zb(i):
        return (0, 0)

    return pl.pallas_call(
        _out_body,
        grid=grid,
        in_specs=[
            pl.BlockSpec((BN, TW), nb),
            pl.BlockSpec(w.shape, zb),
            pl.BlockSpec(bias.shape, zb),
        ],
        out_specs=pl.BlockSpec((BN, OUT), nb),
        out_shape=jax.ShapeDtypeStruct((N, OUT), jnp.float32),
        compiler_params=pltpu.CompilerParams(
            dimension_semantics=("arbitrary",)),
    )(thx, w, bias)


def kernel(h, x, edge_index, edge_attr, params):
    row3 = edge_index[0].reshape(NW, NCHUNK, CHUNK)
    col3 = edge_index[1].reshape(NW, NCHUNK, CHUNK)
    xpad = jnp.pad(x, ((0, 0), (0, XP - 3)))
    z80 = jnp.zeros((N, TW), jnp.float32)

    thx = _embed_tc(h, xpad, params["Wemb"], params["bemb"].reshape(1, HID))
    for p in params["layers"]:
        w1a = p["We1"][:HID]
        w1b = p["We1"][HID:2 * HID]
        w1c = p["We1"][2 * HID:2 * HID + 1]
        w1d = p["We1"][2 * HID + 1:]
        a, b = _sc_gather(thx, row3, col3)
        mt = _edge_tc(
            a, b, edge_attr,
            w1a, w1b, w1c, w1d, p["be1"].reshape(1, HID),
            p["We2"], p["be2"].reshape(1, HID),
            p["Watt"], p["batt"].reshape(1, 1),
            p["Wc1"], p["bc1"].reshape(1, HID), p["Wc2"])
        accp = _sc_scatter(mt, row3, z80)
        thx = _node_tc(
            thx, accp,
            p["Wn1"][:HID], p["Wn1"][HID:],
            p["bn1"].reshape(1, HID), p["Wn2"], p["bn2"].reshape(1, HID),
            p["ln_g"].reshape(1, HID), p["ln_b"].reshape(1, HID))
    return _out_tc(thx, params["Wout"], params["bout"].reshape(1, OUT))


# speedup vs baseline: 1.3323x; 1.3323x over previous
import functools

import jax
import jax.numpy as jnp
from jax import lax
from jax.experimental import pallas as pl
from jax.experimental.pallas import tpu as pltpu
from jax.experimental.pallas import tpu_sc as plsc

N = 10000
E = 320000
IN_DIM = 128
HID = 64
OUT = 64
EDGE_DIM = 16
XP = 16
TW = HID + XP
PW = 128
EPS = 1e-8

NC = 2
NS = 16
NW = NC * NS
EW = E // NW
CHUNK = 80
NCHUNK = EW // CHUNK
KB = 5
NBURST = NCHUNK // KB

SCHUNK = 40
KBS = 2
SNCHUNK = EW // SCHUNK
SNBURST = SNCHUNK // KBS

ROWS_PT = 624
ROWS_LAST = N - 15 * ROWS_PT


def _worker_id():
    return lax.axis_index("s") * NC + lax.axis_index("c")


@functools.cache
def _sc_kernels():
    mesh = plsc.VectorSubcoreMesh(
        core_axis_name="c", subcore_axis_name="s",
        num_cores=NC, num_subcores=NS)

    @functools.partial(
        pl.kernel,
        out_type=(
            jax.ShapeDtypeStruct((E, PW), jnp.float32),
            jax.ShapeDtypeStruct((E, PW), jnp.float32),
        ),
        mesh=mesh,
        scratch_types=(
            pltpu.VMEM((NCHUNK, CHUNK), jnp.int32),
            pltpu.VMEM((NCHUNK, CHUNK), jnp.int32),
            pltpu.VMEM((KB, CHUNK, PW), jnp.float32),
            pltpu.SemaphoreType.DMA,
            pltpu.SemaphoreType.DMA,
        ),
    )
    def sc_gather(thx_hbm, row_hbm, col_hbm, a_hbm, b_hbm,
                  row_v, col_v, gb, gsem, wsem):
        wid = _worker_id()
        base = wid * EW
        pltpu.sync_copy(row_hbm.at[wid], row_v)
        pltpu.sync_copy(col_hbm.at[wid], col_v)

        for idx_v, out_hbm in ((row_v, a_hbm), (col_v, b_hbm)):
            def body(g, carry, idx_v=idx_v, out_hbm=out_hbm):
                j0 = g * KB
                gcp = []
                for b in range(KB):
                    gcp.append(pltpu.async_copy(
                        thx_hbm.at[idx_v.at[j0 + b]], gb.at[b], gsem))
                wcp = []
                for b in range(KB):
                    gcp[b].wait()
                    sl = pl.ds(base + (j0 + b) * CHUNK, CHUNK)
                    wcp.append(pltpu.async_copy(gb.at[b], out_hbm.at[sl], wsem))
                for c in wcp:
                    c.wait()
                return carry

            lax.fori_loop(0, NBURST, body, None)

    @functools.partial(
        pl.kernel,
        out_type=jax.ShapeDtypeStruct((NC, N, PW), jnp.float32),
        mesh=mesh,
        scratch_types=(
            pltpu.VMEM((SNCHUNK, SCHUNK), jnp.int32),
            pltpu.VMEM((KBS, SCHUNK, PW), jnp.float32),
            pltpu.VMEM_SHARED((N, PW), jnp.float32),
            pltpu.SemaphoreType.DMA,
            pltpu.SemaphoreType.DMA,
        ),
    )
    def sc_scatter(mt_hbm, row_hbm, z_hbm, acc_hbm,
                   row_v, mb, acc_sh, lsem, ssem):
        cid = lax.axis_index("c")
        sid = lax.axis_index("s")
        wid = _worker_id()
        base = wid * EW

        @pl.when(sid < 15)
        def _():
            rsl = pl.ds(sid * ROWS_PT, ROWS_PT)
            pltpu.sync_copy(z_hbm.at[rsl], acc_sh.at[rsl])

        @pl.when(sid == 15)
        def _():
            rsl = pl.ds(15 * ROWS_PT, ROWS_LAST)
            pltpu.sync_copy(z_hbm.at[rsl], acc_sh.at[rsl])

        pltpu.sync_copy(row_hbm.at[wid], row_v)
        plsc.subcore_barrier()

        def body(g, carry):
            j0 = g * KBS
            lcp = []
            for b in range(KBS):
                sl = pl.ds(base + (j0 + b) * SCHUNK, SCHUNK)
                lcp.append(pltpu.async_copy(mt_hbm.at[sl], mb.at[b], lsem))
            scp = []
            for b in range(KBS):
                lcp[b].wait()
                scp.append(pltpu.async_copy(
                    mb.at[b], acc_sh.at[row_v.at[j0 + b]], ssem, add=True))
            for c in scp:
                c.wait()
            return carry

        lax.fori_loop(0, SNBURST, body, None)
        plsc.subcore_barrier()

        @pl.when(sid < 15)
        def _():
            rsl = pl.ds(sid * ROWS_PT, ROWS_PT)
            pltpu.sync_copy(acc_sh.at[rsl], acc_hbm.at[cid].at[rsl])

        @pl.when(sid == 15)
        def _():
            rsl = pl.ds(15 * ROWS_PT, ROWS_LAST)
            pltpu.sync_copy(acc_sh.at[rsl], acc_hbm.at[cid].at[rsl])

    return sc_gather, sc_scatter


def _sc_gather(thx, row3, col3):
    return _sc_kernels()[0](thx, row3, col3)


def _sc_scatter(mt, rowS, z):
    return _sc_kernels()[1](mt, rowS, z)


BE = 2000
BN = 1000


def _silu(v):
    return v * jax.nn.sigmoid(v)


def _edge_body(a, b, ea,
               w1a, w1b, w1c, w1d, be1, we2, be2, watt, batt, wc1, bc1, wc2,
               mt_out):
    hi = a[:, :HID]
    hj = b[:, :HID]
    d = a[:, HID:TW] - b[:, HID:TW]
    radial = jnp.sum(d * d, axis=-1, keepdims=True)
    dn = d * lax.rsqrt(jnp.maximum(radial, 1e-24))
    t = jnp.dot(hi, w1a[...], preferred_element_type=jnp.float32)
    t += jnp.dot(hj, w1b[...], preferred_element_type=jnp.float32)
    t += jnp.dot(ea[...], w1d[...], preferred_element_type=jnp.float32)
    t += radial * w1c[...] + be1[...]
    m1 = _silu(t)
    m2 = _silu(jnp.dot(m1, we2[...], preferred_element_type=jnp.float32) + be2[...])
    att = jax.nn.sigmoid(
        jnp.dot(m2, watt[...], preferred_element_type=jnp.float32) + batt[...])
    m = m2 * att
    s = jnp.dot(_silu(jnp.dot(m, wc1[...], preferred_element_type=jnp.float32)
                      + bc1[...]),
                wc2[...], preferred_element_type=jnp.float32)
    tp = dn * s
    colpos = lax.broadcasted_iota(jnp.int32, tp.shape, 1)
    tp = jnp.where(colpos == 3, 1.0, tp)
    mt_out[...] = jnp.concatenate(
        [m, tp, jnp.zeros((m.shape[0], PW - TW), jnp.float32)], axis=1)


def _edge_tc(a, b, ea, w1a, w1b, w1c, w1d, be1, we2, be2,
             watt, batt, wc1, bc1, wc2):
    grid = (E // BE,)

    def eb(i):
        return (i, 0)

    def zb(i):
        return (0, 0)

    full = lambda arr: pl.BlockSpec(arr.shape, zb)
    return pl.pallas_call(
        _edge_body,
        grid=grid,
        in_specs=[
            pl.BlockSpec((BE, PW), eb),
            pl.BlockSpec((BE, PW), eb),
            pl.BlockSpec((BE, EDGE_DIM), eb),
            full(w1a), full(w1b), full(w1c), full(w1d), full(be1),
            full(we2), full(be2), full(watt), full(batt),
            full(wc1), full(bc1), full(wc2),
        ],
        out_specs=pl.BlockSpec((BE, PW), eb),
        out_shape=jax.ShapeDtypeStruct((E, PW), jnp.float32),
        compiler_params=pltpu.CompilerParams(
            dimension_semantics=("arbitrary",)),
    )(a, b, ea, w1a, w1b, w1c, w1d, be1, we2, be2, watt, batt, wc1, bc1, wc2)


def _node_body(thx, accp, wn1a, wn1b, bn1, wn2, bn2, g, b, thx_out):
    h = thx[:, :HID]
    xp = thx[:, HID:TW]
    acc = accp[0] + accp[1]
    agg = acc[:, :HID]
    xs = acc[:, HID:TW]
    cnt = jnp.maximum(xs[:, 3:4], 1.0)
    xnew = xp + xs / cnt
    colpos = lax.broadcasted_iota(jnp.int32, xnew.shape, 1)
    xnew = jnp.where(colpos < 3, xnew, 0.0)
    u = _silu(jnp.dot(h, wn1a[...], preferred_element_type=jnp.float32)
              + jnp.dot(agg, wn1b[...], preferred_element_type=jnp.float32)
              + bn1[...])
    hu = jnp.dot(u, wn2[...], preferred_element_type=jnp.float32) + bn2[...]
    hn = h + hu
    mu = jnp.mean(hn, axis=-1, keepdims=True)
    var = jnp.mean((hn - mu) ** 2, axis=-1, keepdims=True)
    ho = (hn - mu) / jnp.sqrt(var + 1e-5) * g[...] + b[...]
    thx_out[...] = jnp.concatenate(
        [ho, xnew, jnp.zeros((ho.shape[0], PW - TW), jnp.float32)], axis=1)


def _node_tc(thx, accp, wn1a, wn1b, bn1, wn2, bn2, g, b):
    grid = (N // BN,)

    def nb(i):
        return (i, 0)

    def pb(i):
        return (0, i, 0)

    def zb(i):
        return (0, 0)

    full = lambda arr: pl.BlockSpec(arr.shape, zb)
    return pl.pallas_call(
        _node_body,
        grid=grid,
        in_specs=[
            pl.BlockSpec((BN, PW), nb),
            pl.BlockSpec((NC, BN, PW), pb),
            full(wn1a), full(wn1b), full(bn1), full(wn2), full(bn2),
            full(g), full(b),
        ],
        out_specs=pl.BlockSpec((BN, PW), nb),
        out_shape=jax.ShapeDtypeStruct((N, PW), jnp.float32),
        compiler_params=pltpu.CompilerParams(
            dimension_semantics=("arbitrary",)),
    )(thx, accp, wn1a, wn1b, bn1, wn2, bn2, g, b)


def _embed_body(h, xp, w, bias, o):
    hh = jnp.dot(h[...], w[...], preferred_element_type=jnp.float32) + bias[...]
    o[...] = jnp.concatenate(
        [hh, xp[...], jnp.zeros((hh.shape[0], PW - TW), jnp.float32)], axis=1)


def _embed_tc(h, xpad, w, bias):
    grid = (N // BN,)

    def nb(i):
        return (i, 0)

    def zb(i):
        return (0, 0)

    return pl.pallas_call(
        _embed_body,
        grid=grid,
        in_specs=[
            pl.BlockSpec((BN, IN_DIM), nb),
            pl.BlockSpec((BN, XP), nb),
            pl.BlockSpec(w.shape, zb),
            pl.BlockSpec(bias.shape, zb),
        ],
        out_specs=pl.BlockSpec((BN, PW), nb),
        out_shape=jax.ShapeDtypeStruct((N, PW), jnp.float32),
        compiler_params=pltpu.CompilerParams(
            dimension_semantics=("arbitrary",)),
    )(h, xpad, w, bias)


def _out_body(thx, w, bias, o):
    o[...] = jnp.dot(thx[:, :HID], w[...],
                     preferred_element_type=jnp.float32) + bias[...]


def _out_tc(thx, w, bias):
    grid = (N // BN,)

    def nb(i):
        return (i, 0)

    def zb(i):
        return (0, 0)

    return pl.pallas_call(
        _out_body,
        grid=grid,
        in_specs=[
            pl.BlockSpec((BN, PW), nb),
            pl.BlockSpec(w.shape, zb),
            pl.BlockSpec(bias.shape, zb),
        ],
        out_specs=pl.BlockSpec((BN, OUT), nb),
        out_shape=jax.ShapeDtypeStruct((N, OUT), jnp.float32),
        compiler_params=pltpu.CompilerParams(
            dimension_semantics=("arbitrary",)),
    )(thx, w, bias)


def kernel(h, x, edge_index, edge_attr, params):
    row3 = edge_index[0].reshape(NW, NCHUNK, CHUNK)
    col3 = edge_index[1].reshape(NW, NCHUNK, CHUNK)
    rowS = edge_index[0].reshape(NW, SNCHUNK, SCHUNK)
    xpad = jnp.pad(x, ((0, 0), (0, XP - 3)))
    z = jnp.zeros((N, PW), jnp.float32)

    thx = _embed_tc(h, xpad, params["Wemb"], params["bemb"].reshape(1, HID))
    for p in params["layers"]:
        w1a = p["We1"][:HID]
        w1b = p["We1"][HID:2 * HID]
        w1c = p["We1"][2 * HID:2 * HID + 1]
        w1d = p["We1"][2 * HID + 1:]
        a, b = _sc_gather(thx, row3, col3)
        mt = _edge_tc(
            a, b, edge_attr,
            w1a, w1b, w1c, w1d, p["be1"].reshape(1, HID),
            p["We2"], p["be2"].reshape(1, HID),
            p["Watt"], p["batt"].reshape(1, 1),
            p["Wc1"], p["bc1"].reshape(1, HID), p["Wc2"])
        accp = _sc_scatter(mt, rowS, z)
        thx = _node_tc(
            thx, accp,
            p["Wn1"][:HID], p["Wn1"][HID:],
            p["bn1"].reshape(1, HID), p["Wn2"], p["bn2"].reshape(1, HID),
            p["ln_g"].reshape(1, HID), p["ln_b"].reshape(1, HID))
    return _out_tc(thx, params["Wout"], params["bout"].reshape(1, OUT))


# speedup vs baseline: 1.6678x; 1.2518x over previous
import functools

import jax
import jax.numpy as jnp
from jax import lax
from jax.experimental import pallas as pl
from jax.experimental.pallas import tpu as pltpu
from jax.experimental.pallas import tpu_sc as plsc

N = 10000
E = 320000
IN_DIM = 128
HID = 64
OUT = 64
EDGE_DIM = 16
XP = 16
TW = HID + XP
PW = 128
EPS = 1e-8

NC = 2
NS = 16
NW = NC * NS

NSL = 2
ES = E // NSL
EW = ES // NW
CHUNK = 40
NCHUNK = EW // CHUNK
KB = 5
NBURST = NCHUNK // KB

SCHUNK = 40
KBS = 2
SNCHUNK = EW // SCHUNK
SNBURST = SNCHUNK // KBS

ROWS_PT = 624
ROWS_LAST = N - 15 * ROWS_PT


def _worker_id():
    return lax.axis_index("s") * NC + lax.axis_index("c")


@functools.cache
def _sc_kernels():
    mesh = plsc.VectorSubcoreMesh(
        core_axis_name="c", subcore_axis_name="s",
        num_cores=NC, num_subcores=NS)

    @functools.partial(
        pl.kernel,
        out_type=(
            jax.ShapeDtypeStruct((ES, PW), jnp.float32),
            jax.ShapeDtypeStruct((ES, PW), jnp.float32),
        ),
        mesh=mesh,
        scratch_types=(
            pltpu.VMEM((NCHUNK, CHUNK), jnp.int32),
            pltpu.VMEM((NCHUNK, CHUNK), jnp.int32),
            pltpu.VMEM((KB, CHUNK, PW), jnp.float32),
            pltpu.SemaphoreType.DMA,
            pltpu.SemaphoreType.DMA,
        ),
    )
    def sc_gather(thx_hbm, row_hbm, col_hbm, a_hbm, b_hbm,
                  row_v, col_v, gb, gsem, wsem):
        wid = _worker_id()
        base = wid * EW
        pltpu.sync_copy(row_hbm.at[wid], row_v)
        pltpu.sync_copy(col_hbm.at[wid], col_v)

        for idx_v, out_hbm in ((row_v, a_hbm), (col_v, b_hbm)):
            def body(g, carry, idx_v=idx_v, out_hbm=out_hbm):
                j0 = g * KB
                gcp = []
                for b in range(KB):
                    gcp.append(pltpu.async_copy(
                        thx_hbm.at[idx_v.at[j0 + b]], gb.at[b], gsem))
                wcp = []
                for b in range(KB):
                    gcp[b].wait()
                    sl = pl.ds(base + (j0 + b) * CHUNK, CHUNK)
                    wcp.append(pltpu.async_copy(gb.at[b], out_hbm.at[sl], wsem))
                for c in wcp:
                    c.wait()
                return carry

            lax.fori_loop(0, NBURST, body, None)

    @functools.partial(
        pl.kernel,
        out_type=jax.ShapeDtypeStruct((NC, N, PW), jnp.float32),
        mesh=mesh,
        scratch_types=(
            pltpu.VMEM((SNCHUNK, SCHUNK), jnp.int32),
            pltpu.VMEM((KBS, SCHUNK, PW), jnp.float32),
            pltpu.VMEM_SHARED((N, PW), jnp.float32),
            pltpu.SemaphoreType.DMA,
            pltpu.SemaphoreType.DMA,
        ),
    )
    def sc_scatter(mt_hbm, row_hbm, z_hbm, acc_hbm,
                   row_v, mb, acc_sh, lsem, ssem):
        cid = lax.axis_index("c")
        sid = lax.axis_index("s")
        wid = _worker_id()
        base = wid * EW

        @pl.when(sid < 15)
        def _():
            rsl = pl.ds(sid * ROWS_PT, ROWS_PT)
            pltpu.sync_copy(z_hbm.at[rsl], acc_sh.at[rsl])

        @pl.when(sid == 15)
        def _():
            rsl = pl.ds(15 * ROWS_PT, ROWS_LAST)
            pltpu.sync_copy(z_hbm.at[rsl], acc_sh.at[rsl])

        pltpu.sync_copy(row_hbm.at[wid], row_v)
        plsc.subcore_barrier()

        def body(g, carry):
            j0 = g * KBS
            lcp = []
            for b in range(KBS):
                sl = pl.ds(base + (j0 + b) * SCHUNK, SCHUNK)
                lcp.append(pltpu.async_copy(mt_hbm.at[sl], mb.at[b], lsem))
            scp = []
            for b in range(KBS):
                lcp[b].wait()
                scp.append(pltpu.async_copy(
                    mb.at[b], acc_sh.at[row_v.at[j0 + b]], ssem, add=True))
            for c in scp:
                c.wait()
            return carry

        lax.fori_loop(0, SNBURST, body, None)
        for j in range(SNBURST * KBS, SNCHUNK):
            sl = pl.ds(base + j * SCHUNK, SCHUNK)
            pltpu.async_copy(mt_hbm.at[sl], mb.at[0], lsem).wait()
            pltpu.async_copy(
                mb.at[0], acc_sh.at[row_v.at[j]], ssem, add=True).wait()
        plsc.subcore_barrier()

        @pl.when(sid < 15)
        def _():
            rsl = pl.ds(sid * ROWS_PT, ROWS_PT)
            pltpu.sync_copy(acc_sh.at[rsl], acc_hbm.at[cid].at[rsl])

        @pl.when(sid == 15)
        def _():
            rsl = pl.ds(15 * ROWS_PT, ROWS_LAST)
            pltpu.sync_copy(acc_sh.at[rsl], acc_hbm.at[cid].at[rsl])

    return sc_gather, sc_scatter


def _sc_gather(thx, row3, col3):
    return _sc_kernels()[0](thx, row3, col3)


def _sc_scatter(mt, rowS, z):
    return _sc_kernels()[1](mt, rowS, z)


BE = 3200
BN = 1000


def _silu(v):
    return v * jax.nn.sigmoid(v)


def _edge_body(a, b, eat,
               w1a, w1b, w1c, w1d, be1, we2, be2, watt, batt, wc1, bc1, wc2,
               mt_out):
    hi = a[:, :HID]
    hj = b[:, :HID]
    d = a[:, HID:TW] - b[:, HID:TW]
    radial = jnp.sum(d * d, axis=-1, keepdims=True)
    dn = d * lax.rsqrt(jnp.maximum(radial, 1e-24))
    t = jnp.dot(hi, w1a[...], preferred_element_type=jnp.float32)
    t += jnp.dot(hj, w1b[...], preferred_element_type=jnp.float32)
    t += lax.dot_general(eat[...], w1d[...], (((0,), (0,)), ((), ())),
                         preferred_element_type=jnp.float32)
    t += radial * w1c[...] + be1[...]
    m1 = _silu(t)
    m2 = _silu(jnp.dot(m1, we2[...], preferred_element_type=jnp.float32) + be2[...])
    att = jax.nn.sigmoid(
        jnp.dot(m2, watt[...], preferred_element_type=jnp.float32) + batt[...])
    m = m2 * att
    s = jnp.dot(_silu(jnp.dot(m, wc1[...], preferred_element_type=jnp.float32)
                      + bc1[...]),
                wc2[...], preferred_element_type=jnp.float32)
    tp = dn * s
    colpos = lax.broadcasted_iota(jnp.int32, tp.shape, 1)
    tp = jnp.where(colpos == 3, 1.0, tp)
    mt_out[...] = jnp.concatenate(
        [m, tp, jnp.zeros((m.shape[0], PW - TW), jnp.float32)], axis=1)


def _edge_tc(sl, a, b, eat, w1a, w1b, w1c, w1d, be1, we2, be2,
             watt, batt, wc1, bc1, wc2):
    grid = (ES // BE,)
    off = sl * (ES // BE)

    def eb(i):
        return (i, 0)

    def ebt(i):
        return (0, i + off)

    def zb(i):
        return (0, 0)

    full = lambda arr: pl.BlockSpec(arr.shape, zb)
    return pl.pallas_call(
        _edge_body,
        grid=grid,
        in_specs=[
            pl.BlockSpec((BE, PW), eb),
            pl.BlockSpec((BE, PW), eb),
            pl.BlockSpec((EDGE_DIM, BE), ebt),
            full(w1a), full(w1b), full(w1c), full(w1d), full(be1),
            full(we2), full(be2), full(watt), full(batt),
            full(wc1), full(bc1), full(wc2),
        ],
        out_specs=pl.BlockSpec((BE, PW), eb),
        out_shape=jax.ShapeDtypeStruct((ES, PW), jnp.float32),
        compiler_params=pltpu.CompilerParams(
            dimension_semantics=("arbitrary",)),
    )(a, b, eat, w1a, w1b, w1c, w1d, be1, we2, be2, watt, batt, wc1, bc1, wc2)


def _node_body(thx, accp, accq, wn1a, wn1b, bn1, wn2, bn2, g, b, thx_out):
    h = thx[:, :HID]
    xp = thx[:, HID:TW]
    acc = (accp[0] + accp[1]) + (accq[0] + accq[1])
    agg = acc[:, :HID]
    xs = acc[:, HID:TW]
    cnt = jnp.maximum(xs[:, 3:4], 1.0)
    xnew = xp + xs / cnt
    colpos = lax.broadcasted_iota(jnp.int32, xnew.shape, 1)
    xnew = jnp.where(colpos < 3, xnew, 0.0)
    u = _silu(jnp.dot(h, wn1a[...], preferred_element_type=jnp.float32)
              + jnp.dot(agg, wn1b[...], preferred_element_type=jnp.float32)
              + bn1[...])
    hu = jnp.dot(u, wn2[...], preferred_element_type=jnp.float32) + bn2[...]
    hn = h + hu
    mu = jnp.mean(hn, axis=-1, keepdims=True)
    var = jnp.mean((hn - mu) ** 2, axis=-1, keepdims=True)
    ho = (hn - mu) / jnp.sqrt(var + 1e-5) * g[...] + b[...]
    thx_out[...] = jnp.concatenate(
        [ho, xnew, jnp.zeros((ho.shape[0], PW - TW), jnp.float32)], axis=1)


def _node_tc(thx, accp, accq, wn1a, wn1b, bn1, wn2, bn2, g, b):
    grid = (N // BN,)

    def nb(i):
        return (i, 0)

    def pb(i):
        return (0, i, 0)

    def zb(i):
        return (0, 0)

    full = lambda arr: pl.BlockSpec(arr.shape, zb)
    return pl.pallas_call(
        _node_body,
        grid=grid,
        in_specs=[
            pl.BlockSpec((BN, PW), nb),
            pl.BlockSpec((NC, BN, PW), pb),
            pl.BlockSpec((NC, BN, PW), pb),
            full(wn1a), full(wn1b), full(bn1), full(wn2), full(bn2),
            full(g), full(b),
        ],
        out_specs=pl.BlockSpec((BN, PW), nb),
        out_shape=jax.ShapeDtypeStruct((N, PW), jnp.float32),
        compiler_params=pltpu.CompilerParams(
            dimension_semantics=("arbitrary",)),
    )(thx, accp, accq, wn1a, wn1b, bn1, wn2, bn2, g, b)


def _embed_body(h, xp, w, bias, o):
    hh = jnp.dot(h[...], w[...], preferred_element_type=jnp.float32) + bias[...]
    o[...] = jnp.concatenate(
        [hh, xp[...], jnp.zeros((hh.shape[0], PW - TW), jnp.float32)], axis=1)


def _embed_tc(h, xpad, w, bias):
    grid = (N // BN,)

    def nb(i):
        return (i, 0)

    def zb(i):
        return (0, 0)

    return pl.pallas_call(
        _embed_body,
        grid=grid,
        in_specs=[
            pl.BlockSpec((BN, IN_DIM), nb),
            pl.BlockSpec((BN, XP), nb),
            pl.BlockSpec(w.shape, zb),
            pl.BlockSpec(bias.shape, zb),
        ],
        out_specs=pl.BlockSpec((BN, PW), nb),
        out_shape=jax.ShapeDtypeStruct((N, PW), jnp.float32),
        compiler_params=pltpu.CompilerParams(
            dimension_semantics=("arbitrary",)),
    )(h, xpad, w, bias)


def _out_body(thx, w, bias, o):
    o[...] = jnp.dot(thx[:, :HID], w[...],
                     preferred_element_type=jnp.float32) + bias[...]


def _out_tc(thx, w, bias):
    grid = (N // BN,)

    def nb(i):
        return (i, 0)

    def zb(i):
        return (0, 0)

    return pl.pallas_call(
        _out_body,
        grid=grid,
        in_specs=[
            pl.BlockSpec((BN, PW), nb),
            pl.BlockSpec(w.shape, zb),
            pl.BlockSpec(bias.shape, zb),
        ],
        out_specs=pl.BlockSpec((BN, OUT), nb),
        out_shape=jax.ShapeDtypeStruct((N, OUT), jnp.float32),
        compiler_params=pltpu.CompilerParams(
            dimension_semantics=("arbitrary",)),
    )(thx, w, bias)


def kernel(h, x, edge_index, edge_attr, params):
    row4 = edge_index[0].reshape(NSL, NW, NCHUNK, CHUNK)
    col4 = edge_index[1].reshape(NSL, NW, NCHUNK, CHUNK)
    rowS = edge_index[0].reshape(NSL, NW, SNCHUNK, SCHUNK)
    eat = jnp.swapaxes(edge_attr, 0, 1)
    xpad = jnp.pad(x, ((0, 0), (0, XP - 3)))
    z = jnp.zeros((N, PW), jnp.float32)

    thx = _embed_tc(h, xpad, params["Wemb"], params["bemb"].reshape(1, HID))
    for p in params["layers"]:
        w1a = p["We1"][:HID]
        w1b = p["We1"][HID:2 * HID]
        w1c = p["We1"][2 * HID:2 * HID + 1]
        w1d = p["We1"][2 * HID + 1:]
        accs = []
        for sl in range(NSL):
            a, b = _sc_gather(thx, row4[sl], col4[sl])
            mt = _edge_tc(
                sl, a, b, eat,
                w1a, w1b, w1c, w1d, p["be1"].reshape(1, HID),
                p["We2"], p["be2"].reshape(1, HID),
                p["Watt"], p["batt"].reshape(1, 1),
                p["Wc1"], p["bc1"].reshape(1, HID), p["Wc2"])
            accs.append(_sc_scatter(mt, rowS[sl], z))
        thx = _node_tc(
            thx, accs[0], accs[1],
            p["Wn1"][:HID], p["Wn1"][HID:],
            p["bn1"].reshape(1, HID), p["Wn2"], p["bn2"].reshape(1, HID),
            p["ln_g"].reshape(1, HID), p["ln_b"].reshape(1, HID))
    return _out_tc(thx, params["Wout"], params["bout"].reshape(1, OUT))


# speedup vs baseline: 1.7023x; 1.0207x over previous
import functools

import jax
import jax.numpy as jnp
from jax import lax
from jax.experimental import pallas as pl
from jax.experimental.pallas import tpu as pltpu
from jax.experimental.pallas import tpu_sc as plsc

N = 10000
E = 320000
IN_DIM = 128
HID = 64
OUT = 64
EDGE_DIM = 16
XP = 16
TW = HID + XP
PW = 128
EPS = 1e-8

NC = 2
NS = 16
NW = NC * NS

NSL = 5
ES = E // NSL
EW = ES // NW
CHUNK = 40
NCHUNK = EW // CHUNK
KB = 5
NBURST = NCHUNK // KB

SCHUNK = 40
KBS = 2
SNCHUNK = EW // SCHUNK
SNBURST = SNCHUNK // KBS

ROWS_PT = 624
ROWS_LAST = N - 15 * ROWS_PT


def _worker_id():
    return lax.axis_index("s") * NC + lax.axis_index("c")


@functools.cache
def _sc_kernels():
    mesh = plsc.VectorSubcoreMesh(
        core_axis_name="c", subcore_axis_name="s",
        num_cores=NC, num_subcores=NS)

    @functools.partial(
        pl.kernel,
        out_type=(
            jax.ShapeDtypeStruct((ES, PW), jnp.float32),
            jax.ShapeDtypeStruct((ES, PW), jnp.float32),
        ),
        mesh=mesh,
        scratch_types=(
            pltpu.VMEM((NCHUNK, CHUNK), jnp.int32),
            pltpu.VMEM((NCHUNK, CHUNK), jnp.int32),
            pltpu.VMEM((KB, CHUNK, PW), jnp.float32),
            pltpu.SemaphoreType.DMA,
            pltpu.SemaphoreType.DMA,
        ),
    )
    def sc_gather(thx_hbm, row_hbm, col_hbm, a_hbm, b_hbm,
                  row_v, col_v, gb, gsem, wsem):
        wid = _worker_id()
        base = wid * EW
        pltpu.sync_copy(row_hbm.at[wid], row_v)
        pltpu.sync_copy(col_hbm.at[wid], col_v)

        for idx_v, out_hbm in ((row_v, a_hbm), (col_v, b_hbm)):
            def body(g, carry, idx_v=idx_v, out_hbm=out_hbm):
                j0 = g * KB
                gcp = []
                for b in range(KB):
                    gcp.append(pltpu.async_copy(
                        thx_hbm.at[idx_v.at[j0 + b]], gb.at[b], gsem))
                wcp = []
                for b in range(KB):
                    gcp[b].wait()
                    sl = pl.ds(base + (j0 + b) * CHUNK, CHUNK)
                    wcp.append(pltpu.async_copy(gb.at[b], out_hbm.at[sl], wsem))
                for c in wcp:
                    c.wait()
                return carry

            lax.fori_loop(0, NBURST, body, None)

    @functools.partial(
        pl.kernel,
        out_type=jax.ShapeDtypeStruct((NC, N, PW), jnp.float32),
        mesh=mesh,
        scratch_types=(
            pltpu.VMEM((SNCHUNK, SCHUNK), jnp.int32),
            pltpu.VMEM((KBS, SCHUNK, PW), jnp.float32),
            pltpu.VMEM_SHARED((N, PW), jnp.float32),
            pltpu.SemaphoreType.DMA,
            pltpu.SemaphoreType.DMA,
        ),
    )
    def sc_scatter(mt_hbm, row_hbm, z_hbm, acc_hbm,
                   row_v, mb, acc_sh, lsem, ssem):
        cid = lax.axis_index("c")
        sid = lax.axis_index("s")
        wid = _worker_id()
        base = wid * EW

        @pl.when(sid < 15)
        def _():
            rsl = pl.ds(sid * ROWS_PT, ROWS_PT)
            pltpu.sync_copy(z_hbm.at[rsl], acc_sh.at[rsl])

        @pl.when(sid == 15)
        def _():
            rsl = pl.ds(15 * ROWS_PT, ROWS_LAST)
            pltpu.sync_copy(z_hbm.at[rsl], acc_sh.at[rsl])

        pltpu.sync_copy(row_hbm.at[wid], row_v)
        plsc.subcore_barrier()

        def body(g, carry):
            j0 = g * KBS
            lcp = []
            for b in range(KBS):
                sl = pl.ds(base + (j0 + b) * SCHUNK, SCHUNK)
                lcp.append(pltpu.async_copy(mt_hbm.at[sl], mb.at[b], lsem))
            scp = []
            for b in range(KBS):
                lcp[b].wait()
                scp.append(pltpu.async_copy(
                    mb.at[b], acc_sh.at[row_v.at[j0 + b]], ssem, add=True))
            for c in scp:
                c.wait()
            return carry

        lax.fori_loop(0, SNBURST, body, None)
        for j in range(SNBURST * KBS, SNCHUNK):
            sl = pl.ds(base + j * SCHUNK, SCHUNK)
            pltpu.async_copy(mt_hbm.at[sl], mb.at[0], lsem).wait()
            pltpu.async_copy(
                mb.at[0], acc_sh.at[row_v.at[j]], ssem, add=True).wait()
        plsc.subcore_barrier()

        @pl.when(sid < 15)
        def _():
            rsl = pl.ds(sid * ROWS_PT, ROWS_PT)
            pltpu.sync_copy(acc_sh.at[rsl], acc_hbm.at[cid].at[rsl])

        @pl.when(sid == 15)
        def _():
            rsl = pl.ds(15 * ROWS_PT, ROWS_LAST)
            pltpu.sync_copy(acc_sh.at[rsl], acc_hbm.at[cid].at[rsl])

    return sc_gather, sc_scatter


def _sc_gather(thx, row3, col3):
    return _sc_kernels()[0](thx, row3, col3)


def _sc_scatter(mt, rowS, z):
    return _sc_kernels()[1](mt, rowS, z)


BE = 3200
BN = 1000


def _silu(v):
    return v * jax.nn.sigmoid(v)


def _edge_body(a, b, eat,
               w1a, w1b, w1c, w1d, be1, we2, be2, watt, batt, wc1, bc1, wc2,
               mt_out):
    hi = a[:, :HID]
    hj = b[:, :HID]
    d = a[:, HID:TW] - b[:, HID:TW]
    radial = jnp.sum(d * d, axis=-1, keepdims=True)
    dn = d * lax.rsqrt(jnp.maximum(radial, 1e-24))
    t = jnp.dot(hi, w1a[...], preferred_element_type=jnp.float32)
    t += jnp.dot(hj, w1b[...], preferred_element_type=jnp.float32)
    t += lax.dot_general(eat[...], w1d[...], (((0,), (0,)), ((), ())),
                         preferred_element_type=jnp.float32)
    t += radial * w1c[...] + be1[...]
    m1 = _silu(t)
    m2 = _silu(jnp.dot(m1, we2[...], preferred_element_type=jnp.float32) + be2[...])
    att = jax.nn.sigmoid(
        jnp.dot(m2, watt[...], preferred_element_type=jnp.float32) + batt[...])
    m = m2 * att
    s = jnp.dot(_silu(jnp.dot(m, wc1[...], preferred_element_type=jnp.float32)
                      + bc1[...]),
                wc2[...], preferred_element_type=jnp.float32)
    tp = dn * s
    colpos = lax.broadcasted_iota(jnp.int32, tp.shape, 1)
    tp = jnp.where(colpos == 3, 1.0, tp)
    mt_out[...] = jnp.concatenate(
        [m, tp, jnp.zeros((m.shape[0], PW - TW), jnp.float32)], axis=1)


def _edge_tc(sl, a, b, eat, w1a, w1b, w1c, w1d, be1, we2, be2,
             watt, batt, wc1, bc1, wc2):
    grid = (ES // BE,)
    off = sl * (ES // BE)

    def eb(i):
        return (i, 0)

    def ebt(i):
        return (0, i + off)

    def zb(i):
        return (0, 0)

    full = lambda arr: pl.BlockSpec(arr.shape, zb)
    return pl.pallas_call(
        _edge_body,
        grid=grid,
        in_specs=[
            pl.BlockSpec((BE, PW), eb),
            pl.BlockSpec((BE, PW), eb),
            pl.BlockSpec((EDGE_DIM, BE), ebt),
            full(w1a), full(w1b), full(w1c), full(w1d), full(be1),
            full(we2), full(be2), full(watt), full(batt),
            full(wc1), full(bc1), full(wc2),
        ],
        out_specs=pl.BlockSpec((BE, PW), eb),
        out_shape=jax.ShapeDtypeStruct((ES, PW), jnp.float32),
        compiler_params=pltpu.CompilerParams(
            dimension_semantics=("arbitrary",)),
    )(a, b, eat, w1a, w1b, w1c, w1d, be1, we2, be2, watt, batt, wc1, bc1, wc2)


def _node_body(thx, *rest):
    (*accs, wn1a, wn1b, bn1, wn2, bn2, g, b, thx_out) = rest
    h = thx[:, :HID]
    xp = thx[:, HID:TW]
    acc = sum(a[0] + a[1] for a in accs)
    agg = acc[:, :HID]
    xs = acc[:, HID:TW]
    cnt = jnp.maximum(xs[:, 3:4], 1.0)
    xnew = xp + xs / cnt
    colpos = lax.broadcasted_iota(jnp.int32, xnew.shape, 1)
    xnew = jnp.where(colpos < 3, xnew, 0.0)
    u = _silu(jnp.dot(h, wn1a[...], preferred_element_type=jnp.float32)
              + jnp.dot(agg, wn1b[...], preferred_element_type=jnp.float32)
              + bn1[...])
    hu = jnp.dot(u, wn2[...], preferred_element_type=jnp.float32) + bn2[...]
    hn = h + hu
    mu = jnp.mean(hn, axis=-1, keepdims=True)
    var = jnp.mean((hn - mu) ** 2, axis=-1, keepdims=True)
    ho = (hn - mu) / jnp.sqrt(var + 1e-5) * g[...] + b[...]
    thx_out[...] = jnp.concatenate(
        [ho, xnew, jnp.zeros((ho.shape[0], PW - TW), jnp.float32)], axis=1)


def _node_tc(thx, accs, wn1a, wn1b, bn1, wn2, bn2, g, b):
    grid = (N // BN,)

    def nb(i):
        return (i, 0)

    def pb(i):
        return (0, i, 0)

    def zb(i):
        return (0, 0)

    full = lambda arr: pl.BlockSpec(arr.shape, zb)
    return pl.pallas_call(
        _node_body,
        grid=grid,
        in_specs=[
            pl.BlockSpec((BN, PW), nb),
            *[pl.BlockSpec((NC, BN, PW), pb) for _ in accs],
            full(wn1a), full(wn1b), full(bn1), full(wn2), full(bn2),
            full(g), full(b),
        ],
        out_specs=pl.BlockSpec((BN, PW), nb),
        out_shape=jax.ShapeDtypeStruct((N, PW), jnp.float32),
        compiler_params=pltpu.CompilerParams(
            dimension_semantics=("arbitrary",)),
    )(thx, *accs, wn1a, wn1b, bn1, wn2, bn2, g, b)


def _embed_body(h, xp, w, bias, o):
    hh = jnp.dot(h[...], w[...], preferred_element_type=jnp.float32) + bias[...]
    o[...] = jnp.concatenate(
        [hh, xp[...], jnp.zeros((hh.shape[0], PW - TW), jnp.float32)], axis=1)


def _embed_tc(h, xpad, w, bias):
    grid = (N // BN,)

    def nb(i):
        return (i, 0)

    def zb(i):
        return (0, 0)

    return pl.pallas_call(
        _embed_body,
        grid=grid,
        in_specs=[
            pl.BlockSpec((BN, IN_DIM), nb),
            pl.BlockSpec((BN, XP), nb),
            pl.BlockSpec(w.shape, zb),
            pl.BlockSpec(bias.shape, zb),
        ],
        out_specs=pl.BlockSpec((BN, PW), nb),
        out_shape=jax.ShapeDtypeStruct((N, PW), jnp.float32),
        compiler_params=pltpu.CompilerParams(
            dimension_semantics=("arbitrary",)),
    )(h, xpad, w, bias)


def _out_body(thx, w, bias, o):
    o[...] = jnp.dot(thx[:, :HID], w[...],
                     preferred_element_type=jnp.float32) + bias[...]


def _out_tc(thx, w, bias):
    grid = (N // BN,)

    def nb(i):
        return (i, 0)

    def zb(i):
        return (0, 0)

    return pl.pallas_call(
        _out_body,
        grid=grid,
        in_specs=[
            pl.BlockSpec((BN, PW), nb),
            pl.BlockSpec(w.shape, zb),
            pl.BlockSpec(bias.shape, zb),
        ],
        out_specs=pl.BlockSpec((BN, OUT), nb),
        out_shape=jax.ShapeDtypeStruct((N, OUT), jnp.float32),
        compiler_params=pltpu.CompilerParams(
            dimension_semantics=("arbitrary",)),
    )(thx, w, bias)


def kernel(h, x, edge_index, edge_attr, params):
    row4 = edge_index[0].reshape(NSL, NW, NCHUNK, CHUNK)
    col4 = edge_index[1].reshape(NSL, NW, NCHUNK, CHUNK)
    rowS = edge_index[0].reshape(NSL, NW, SNCHUNK, SCHUNK)
    eat = jnp.swapaxes(edge_attr, 0, 1)
    xpad = jnp.pad(x, ((0, 0), (0, XP - 3)))
    z = jnp.zeros((N, PW), jnp.float32)

    thx = _embed_tc(h, xpad, params["Wemb"], params["bemb"].reshape(1, HID))
    for p in params["layers"]:
        w1a = p["We1"][:HID]
        w1b = p["We1"][HID:2 * HID]
        w1c = p["We1"][2 * HID:2 * HID + 1]
        w1d = p["We1"][2 * HID + 1:]
        accs = []
        for sl in range(NSL):
            a, b = _sc_gather(thx, row4[sl], col4[sl])
            mt = _edge_tc(
                sl, a, b, eat,
                w1a, w1b, w1c, w1d, p["be1"].reshape(1, HID),
                p["We2"], p["be2"].reshape(1, HID),
                p["Watt"], p["batt"].reshape(1, 1),
                p["Wc1"], p["bc1"].reshape(1, HID), p["Wc2"])
            accs.append(_sc_scatter(mt, rowS[sl], z))
        thx = _node_tc(
            thx, accs,
            p["Wn1"][:HID], p["Wn1"][HID:],
            p["bn1"].reshape(1, HID), p["Wn2"], p["bn2"].reshape(1, HID),
            p["ln_g"].reshape(1, HID), p["ln_b"].reshape(1, HID))
    return _out_tc(thx, params["Wout"], params["bout"].reshape(1, OUT))


# speedup vs baseline: 1.8866x; 1.1082x over previous
import functools

import jax
import jax.numpy as jnp
from jax import lax
from jax.experimental import pallas as pl
from jax.experimental.pallas import tpu as pltpu
from jax.experimental.pallas import tpu_sc as plsc

N = 10000
E = 320000
IN_DIM = 128
HID = 64
OUT = 64
EDGE_DIM = 16
XP = 16
TW = HID + XP
PW = 128
EPS = 1e-8

NC = 2
NS = 16
NW = NC * NS

NSL = 5
ES = E // NSL
EW = ES // NW
CHUNK = 80
NCHUNK = EW // CHUNK
KB = 5
NBURST = NCHUNK // KB

SCHUNK = 40
KBS = 5
SNCHUNK = EW // SCHUNK
SNBURST = SNCHUNK // KBS

ROWS_PT = 624
ROWS_LAST = N - 15 * ROWS_PT


def _worker_id():
    return lax.axis_index("s") * NC + lax.axis_index("c")


@functools.cache
def _sc_kernels():
    mesh = plsc.VectorSubcoreMesh(
        core_axis_name="c", subcore_axis_name="s",
        num_cores=NC, num_subcores=NS)

    @functools.partial(
        pl.kernel,
        out_type=(
            jax.ShapeDtypeStruct((ES, PW), jnp.float32),
            jax.ShapeDtypeStruct((ES, PW), jnp.float32),
        ),
        mesh=mesh,
        scratch_types=(
            pltpu.VMEM((NCHUNK, CHUNK), jnp.int32),
            pltpu.VMEM((NCHUNK, CHUNK), jnp.int32),
            pltpu.VMEM((KB, CHUNK, PW), jnp.float32),
            pltpu.SemaphoreType.DMA,
            pltpu.SemaphoreType.DMA,
        ),
    )
    def sc_gather(thx_hbm, row_hbm, col_hbm, a_hbm, b_hbm,
                  row_v, col_v, gb, gsem, wsem):
        wid = _worker_id()
        base = wid * EW
        pltpu.sync_copy(row_hbm.at[wid], row_v)
        pltpu.sync_copy(col_hbm.at[wid], col_v)

        for idx_v, out_hbm in ((row_v, a_hbm), (col_v, b_hbm)):
            def body(g, carry, idx_v=idx_v, out_hbm=out_hbm):
                j0 = g * KB
                gcp = []
                for b in range(KB):
                    gcp.append(pltpu.async_copy(
                        thx_hbm.at[idx_v.at[j0 + b]], gb.at[b], gsem))
                wcp = []
                for b in range(KB):
                    gcp[b].wait()
                    sl = pl.ds(base + (j0 + b) * CHUNK, CHUNK)
                    wcp.append(pltpu.async_copy(gb.at[b], out_hbm.at[sl], wsem))
                for c in wcp:
                    c.wait()
                return carry

            lax.fori_loop(0, NBURST, body, None)

    @functools.partial(
        pl.kernel,
        out_type=jax.ShapeDtypeStruct((NC, N, PW), jnp.float32),
        mesh=mesh,
        scratch_types=(
            pltpu.VMEM((SNCHUNK, SCHUNK), jnp.int32),
            pltpu.VMEM((KBS, SCHUNK, PW), jnp.float32),
            pltpu.VMEM_SHARED((N, PW), jnp.float32),
            pltpu.SemaphoreType.DMA,
            pltpu.SemaphoreType.DMA,
        ),
    )
    def sc_scatter(mt_hbm, row_hbm, z_hbm, acc_hbm,
                   row_v, mb, acc_sh, lsem, ssem):
        cid = lax.axis_index("c")
        sid = lax.axis_index("s")
        wid = _worker_id()
        base = wid * EW

        @pl.when(sid < 15)
        def _():
            rsl = pl.ds(sid * ROWS_PT, ROWS_PT)
            pltpu.sync_copy(z_hbm.at[rsl], acc_sh.at[rsl])

        @pl.when(sid == 15)
        def _():
            rsl = pl.ds(15 * ROWS_PT, ROWS_LAST)
            pltpu.sync_copy(z_hbm.at[rsl], acc_sh.at[rsl])

        pltpu.sync_copy(row_hbm.at[wid], row_v)
        plsc.subcore_barrier()

        def body(g, carry):
            j0 = g * KBS
            lcp = []
            for b in range(KBS):
                sl = pl.ds(base + (j0 + b) * SCHUNK, SCHUNK)
                lcp.append(pltpu.async_copy(mt_hbm.at[sl], mb.at[b], lsem))
            scp = []
            for b in range(KBS):
                lcp[b].wait()
                scp.append(pltpu.async_copy(
                    mb.at[b], acc_sh.at[row_v.at[j0 + b]], ssem, add=True))
            for c in scp:
                c.wait()
            return carry

        lax.fori_loop(0, SNBURST, body, None)
        for j in range(SNBURST * KBS, SNCHUNK):
            sl = pl.ds(base + j * SCHUNK, SCHUNK)
            pltpu.async_copy(mt_hbm.at[sl], mb.at[0], lsem).wait()
            pltpu.async_copy(
                mb.at[0], acc_sh.at[row_v.at[j]], ssem, add=True).wait()
        plsc.subcore_barrier()

        @pl.when(sid < 15)
        def _():
            rsl = pl.ds(sid * ROWS_PT, ROWS_PT)
            pltpu.sync_copy(acc_sh.at[rsl], acc_hbm.at[cid].at[rsl])

        @pl.when(sid == 15)
        def _():
            rsl = pl.ds(15 * ROWS_PT, ROWS_LAST)
            pltpu.sync_copy(acc_sh.at[rsl], acc_hbm.at[cid].at[rsl])

    return sc_gather, sc_scatter


def _sc_gather(thx, row3, col3):
    return _sc_kernels()[0](thx, row3, col3)


def _sc_scatter(mt, rowS, z):
    return _sc_kernels()[1](mt, rowS, z)


BE = 3200
BN = 1000


def _silu(v):
    return v * jax.nn.sigmoid(v)


def _edge_body(a, b, eat,
               w1a, w1b, w1c, w1d, be1, we2, be2, watt, batt, wc1, bc1, wc2,
               mt_out):
    hi = a[:, :HID]
    hj = b[:, :HID]
    d = a[:, HID:TW] - b[:, HID:TW]
    radial = jnp.sum(d * d, axis=-1, keepdims=True)
    dn = d * lax.rsqrt(jnp.maximum(radial, 1e-24))
    t = jnp.dot(hi, w1a[...], preferred_element_type=jnp.float32)
    t += jnp.dot(hj, w1b[...], preferred_element_type=jnp.float32)
    t += lax.dot_general(eat[...], w1d[...], (((0,), (0,)), ((), ())),
                         preferred_element_type=jnp.float32)
    t += radial * w1c[...] + be1[...]
    m1 = _silu(t)
    m2 = _silu(jnp.dot(m1, we2[...], preferred_element_type=jnp.float32) + be2[...])
    att = jax.nn.sigmoid(
        jnp.dot(m2, watt[...], preferred_element_type=jnp.float32) + batt[...])
    m = m2 * att
    s = jnp.dot(_silu(jnp.dot(m, wc1[...], preferred_element_type=jnp.float32)
                      + bc1[...]),
                wc2[...], preferred_element_type=jnp.float32)
    tp = dn * s
    colpos = lax.broadcasted_iota(jnp.int32, tp.shape, 1)
    tp = jnp.where(colpos == 3, 1.0, tp)
    mt_out[...] = jnp.concatenate(
        [m, tp, jnp.zeros((m.shape[0], PW - TW), jnp.float32)], axis=1)


def _edge_tc(sl, a, b, eat, w1a, w1b, w1c, w1d, be1, we2, be2,
             watt, batt, wc1, bc1, wc2):
    grid = (ES // BE,)
    off = sl * (ES // BE)

    def eb(i):
        return (i, 0)

    def ebt(i):
        return (0, i + off)

    def zb(i):
        return (0, 0)

    full = lambda arr: pl.BlockSpec(arr.shape, zb)
    return pl.pallas_call(
        _edge_body,
        grid=grid,
        in_specs=[
            pl.BlockSpec((BE, PW), eb),
            pl.BlockSpec((BE, PW), eb),
            pl.BlockSpec((EDGE_DIM, BE), ebt),
            full(w1a), full(w1b), full(w1c), full(w1d), full(be1),
            full(we2), full(be2), full(watt), full(batt),
            full(wc1), full(bc1), full(wc2),
        ],
        out_specs=pl.BlockSpec((BE, PW), eb),
        out_shape=jax.ShapeDtypeStruct((ES, PW), jnp.float32),
        compiler_params=pltpu.CompilerParams(
            dimension_semantics=("arbitrary",)),
    )(a, b, eat, w1a, w1b, w1c, w1d, be1, we2, be2, watt, batt, wc1, bc1, wc2)


def _node_body(thx, *rest):
    (*accs, wn1a, wn1b, bn1, wn2, bn2, g, b, thx_out) = rest
    h = thx[:, :HID]
    xp = thx[:, HID:TW]
    acc = sum(a[0] + a[1] for a in accs)
    agg = acc[:, :HID]
    xs = acc[:, HID:TW]
    cnt = jnp.maximum(xs[:, 3:4], 1.0)
    xnew = xp + xs / cnt
    colpos = lax.broadcasted_iota(jnp.int32, xnew.shape, 1)
    xnew = jnp.where(colpos < 3, xnew, 0.0)
    u = _silu(jnp.dot(h, wn1a[...], preferred_element_type=jnp.float32)
              + jnp.dot(agg, wn1b[...], preferred_element_type=jnp.float32)
              + bn1[...])
    hu = jnp.dot(u, wn2[...], preferred_element_type=jnp.float32) + bn2[...]
    hn = h + hu
    mu = jnp.mean(hn, axis=-1, keepdims=True)
    var = jnp.mean((hn - mu) ** 2, axis=-1, keepdims=True)
    ho = (hn - mu) / jnp.sqrt(var + 1e-5) * g[...] + b[...]
    thx_out[...] = jnp.concatenate(
        [ho, xnew, jnp.zeros((ho.shape[0], PW - TW), jnp.float32)], axis=1)


def _node_tc(thx, accs, wn1a, wn1b, bn1, wn2, bn2, g, b):
    grid = (N // BN,)

    def nb(i):
        return (i, 0)

    def pb(i):
        return (0, i, 0)

    def zb(i):
        return (0, 0)

    full = lambda arr: pl.BlockSpec(arr.shape, zb)
    return pl.pallas_call(
        _node_body,
        grid=grid,
        in_specs=[
            pl.BlockSpec((BN, PW), nb),
            *[pl.BlockSpec((NC, BN, PW), pb) for _ in accs],
            full(wn1a), full(wn1b), full(bn1), full(wn2), full(bn2),
            full(g), full(b),
        ],
        out_specs=pl.BlockSpec((BN, PW), nb),
        out_shape=jax.ShapeDtypeStruct((N, PW), jnp.float32),
        compiler_params=pltpu.CompilerParams(
            dimension_semantics=("arbitrary",)),
    )(thx, *accs, wn1a, wn1b, bn1, wn2, bn2, g, b)


def _embed_body(h, xp, w, bias, o):
    hh = jnp.dot(h[...], w[...], preferred_element_type=jnp.float32) + bias[...]
    o[...] = jnp.concatenate(
        [hh, xp[...], jnp.zeros((hh.shape[0], PW - TW), jnp.float32)], axis=1)


def _embed_tc(h, xpad, w, bias):
    grid = (N // BN,)

    def nb(i):
        return (i, 0)

    def zb(i):
        return (0, 0)

    return pl.pallas_call(
        _embed_body,
        grid=grid,
        in_specs=[
            pl.BlockSpec((BN, IN_DIM), nb),
            pl.BlockSpec((BN, XP), nb),
            pl.BlockSpec(w.shape, zb),
            pl.BlockSpec(bias.shape, zb),
        ],
        out_specs=pl.BlockSpec((BN, PW), nb),
        out_shape=jax.ShapeDtypeStruct((N, PW), jnp.float32),
        compiler_params=pltpu.CompilerParams(
            dimension_semantics=("arbitrary",)),
    )(h, xpad, w, bias)


def _out_body(thx, w, bias, o):
    o[...] = jnp.dot(thx[:, :HID], w[...],
                     preferred_element_type=jnp.float32) + bias[...]


def _out_tc(thx, w, bias):
    grid = (N // BN,)

    def nb(i):
        return (i, 0)

    def zb(i):
        return (0, 0)

    return pl.pallas_call(
        _out_body,
        grid=grid,
        in_specs=[
            pl.BlockSpec((BN, PW), nb),
            pl.BlockSpec(w.shape, zb),
            pl.BlockSpec(bias.shape, zb),
        ],
        out_specs=pl.BlockSpec((BN, OUT), nb),
        out_shape=jax.ShapeDtypeStruct((N, OUT), jnp.float32),
        compiler_params=pltpu.CompilerParams(
            dimension_semantics=("arbitrary",)),
    )(thx, w, bias)


def kernel(h, x, edge_index, edge_attr, params):
    row4 = edge_index[0].reshape(NSL, NW, NCHUNK, CHUNK)
    col4 = edge_index[1].reshape(NSL, NW, NCHUNK, CHUNK)
    rowS = edge_index[0].reshape(NSL, NW, SNCHUNK, SCHUNK)
    eat = jnp.swapaxes(edge_attr, 0, 1)
    xpad = jnp.pad(x, ((0, 0), (0, XP - 3)))
    z = jnp.zeros((N, PW), jnp.float32)

    thx = _embed_tc(h, xpad, params["Wemb"], params["bemb"].reshape(1, HID))
    for p in params["layers"]:
        w1a = p["We1"][:HID]
        w1b = p["We1"][HID:2 * HID]
        w1c = p["We1"][2 * HID:2 * HID + 1]
        w1d = p["We1"][2 * HID + 1:]
        accs = []
        for sl in range(NSL):
            a, b = _sc_gather(thx, row4[sl], col4[sl])
            mt = _edge_tc(
                sl, a, b, eat,
                w1a, w1b, w1c, w1d, p["be1"].reshape(1, HID),
                p["We2"], p["be2"].reshape(1, HID),
                p["Watt"], p["batt"].reshape(1, 1),
                p["Wc1"], p["bc1"].reshape(1, HID), p["Wc2"])
            accs.append(_sc_scatter(mt, rowS[sl], z))
        thx = _node_tc(
            thx, accs,
            p["Wn1"][:HID], p["Wn1"][HID:],
            p["bn1"].reshape(1, HID), p["Wn2"], p["bn2"].reshape(1, HID),
            p["ln_g"].reshape(1, HID), p["ln_b"].reshape(1, HID))
    return _out_tc(thx, params["Wout"], params["bout"].reshape(1, OUT))
